# baseline (speedup 1.0000x reference)
import functools

import jax
import jax.numpy as jnp
from jax import lax
from jax.experimental import pallas as pl
from jax.experimental.pallas import tpu as pltpu
from jax.experimental.pallas import tpu_sc as plsc

_NC = 2
_NS = 16
_NW = _NC * _NS
_GR = 16


@functools.lru_cache(maxsize=None)
def _make_circulant_kernel(H, S):
  rows_per_w = S // _NW
  n_groups = rows_per_w // _GR
  n_units = H * n_groups
  mesh = plsc.VectorSubcoreMesh(core_axis_name="c", subcore_axis_name="s")

  @functools.partial(
      pl.kernel,
      mesh=mesh,
      out_type=jax.ShapeDtypeStruct((H, S, S), jnp.float32),
      scratch_types=[
          pltpu.VMEM((_GR, S), jnp.float32),
          pltpu.VMEM((_GR, S), jnp.float32),
          pltpu.VMEM((_GR, S), jnp.float32),
          pltpu.SemaphoreType.DMA,
          pltpu.SemaphoreType.DMA,
          pltpu.SemaphoreType.DMA,
      ],
  )
  def k(e128_hbm, out_hbm, b0, b1, b2, s0, s1, s2):
    bufs = (b0, b1, b2)
    sems = (s0, s1, s2)
    wid = lax.axis_index("s") * _NC + lax.axis_index("c")
    i0 = pl.multiple_of(wid * rows_per_w, 8)
    u0 = pl.multiple_of(lax.rem(wid, 2) * rows_per_w, 8)
    i_blk = pl.multiple_of((wid // 2) * 128, 128)
    col = pl.multiple_of(S - 128 - i_blk, 128)

    def issue_load(n, buf, sem):
      h, g = divmod(n, n_groups)
      pltpu.async_copy(
          e128_hbm.at[h, pl.ds(u0 + _GR * g, _GR), pl.ds(col, S)], buf, sem)

    def issue_store(n, buf, sem):
      h, g = divmod(n, n_groups)
      pltpu.async_copy(buf, out_hbm.at[h, pl.ds(i0 + _GR * g, _GR)], sem)

    def wait(buf, sem):
      pltpu.make_async_copy(e128_hbm.at[0, pl.ds(0, _GR), pl.ds(0, S)],
                            buf, sem).wait()

    issue_load(0, bufs[0], sems[0])
    issue_load(1, bufs[1], sems[1])
    for n in range(n_units):
      b, s = bufs[n % 3], sems[n % 3]
      wait(b, s)
      issue_store(n, b, s)
      if n + 2 < n_units:
        nb, ns = bufs[(n + 2) % 3], sems[(n + 2) % 3]
        if n >= 1:
          wait(nb, ns)
        issue_load(n + 2, nb, ns)
    wait(bufs[(n_units - 3) % 3], sems[(n_units - 3) % 3])
    wait(bufs[(n_units - 2) % 3], sems[(n_units - 2) % 3])
    wait(bufs[(n_units - 1) % 3], sems[(n_units - 1) % 3])

  return k


def kernel(seq_len, bias_table):
  del seq_len
  S, H = bias_table.shape
  r = jnp.flip(bias_table, axis=0)
  big = jnp.concatenate([r, r, r], axis=0)
  e128 = jnp.stack([big[127 - u:127 - u + 2 * S] for u in range(128)], axis=0)
  e128 = jnp.transpose(e128, (2, 0, 1))
  return _make_circulant_kernel(H, S)(e128)


# speedup vs baseline: 1.9128x; 1.9128x over previous
import functools

import jax
import jax.numpy as jnp
from jax import lax
from jax.experimental import pallas as pl
from jax.experimental.pallas import tpu as pltpu
from jax.experimental.pallas import tpu_sc as plsc

_NC = 2
_NS = 16
_NW = _NC * _NS
_GR = 8
_NBUF = 6
_PRIME = 3


@functools.lru_cache(maxsize=None)
def _make_circulant_kernel(H, S):
  rows_per_w = S // _NW
  n_groups = rows_per_w // _GR
  n_units = H * n_groups
  mesh = plsc.VectorSubcoreMesh(core_axis_name="c", subcore_axis_name="s")

  @functools.partial(
      pl.kernel,
      mesh=mesh,
      out_type=jax.ShapeDtypeStruct((H, S, S), jnp.float32),
      scratch_types=[pltpu.VMEM((_GR, S), jnp.float32)] * _NBUF
      + [pltpu.SemaphoreType.DMA] * _NBUF,
  )
  def k(e128_hbm, out_hbm, *scratch):
    bufs, sems = scratch[:_NBUF], scratch[_NBUF:]
    wid = lax.axis_index("s") * _NC + lax.axis_index("c")
    i0 = pl.multiple_of(wid * rows_per_w, 8)
    u0 = pl.multiple_of(lax.rem(wid, 2) * rows_per_w, 8)
    i_blk = pl.multiple_of((wid // 2) * 128, 128)
    col = pl.multiple_of(S - 128 - i_blk, 128)

    def issue_load(n):
      h, g = divmod(n, n_groups)
      pltpu.async_copy(
          e128_hbm.at[h, pl.ds(u0 + _GR * g, _GR), pl.ds(col, S)],
          bufs[n % _NBUF], sems[n % _NBUF])

    def issue_store(n):
      h, g = divmod(n, n_groups)
      pltpu.async_copy(bufs[n % _NBUF],
                       out_hbm.at[h, pl.ds(i0 + _GR * g, _GR)],
                       sems[n % _NBUF])

    def wait(n):
      pltpu.make_async_copy(e128_hbm.at[0, pl.ds(0, _GR), pl.ds(0, S)],
                            bufs[n % _NBUF], sems[n % _NBUF]).wait()

    for n in range(_PRIME):
      issue_load(n)
    for n in range(n_units):
      wait(n)
      issue_store(n)
      m = n + _PRIME
      if m < n_units:
        if m - _NBUF >= 0:
          wait(m - _NBUF)
        issue_load(m)
    for n in range(max(0, n_units - _NBUF), n_units):
      wait(n)

  return k


def _expand_body(t_ref, out_ref):
  s = t_ref.shape[-1]
  b = pltpu.roll(t_ref[0], s - 127, axis=1)
  d = 1
  while d < 128:
    b = jnp.concatenate([b, pltpu.roll(b, d, axis=1)], axis=0)
    d *= 2
  out_ref[...] = jnp.concatenate([b, b], axis=1)[None]


@functools.lru_cache(maxsize=None)
def _make_expand_kernel(H, S):
  return pl.pallas_call(
      _expand_body,
      grid=(H,),
      in_specs=[pl.BlockSpec((1, 1, S), lambda h: (h, 0, 0))],
      out_specs=pl.BlockSpec((1, 128, 2 * S), lambda h: (h, 0, 0)),
      out_shape=jax.ShapeDtypeStruct((H, 128, 2 * S), jnp.float32),
  )


def kernel(seq_len, bias_table):
  del seq_len
  S, H = bias_table.shape
  t = jnp.flip(bias_table, axis=0).T[:, None, :]
  e128 = _make_expand_kernel(H, S)(t)
  return _make_circulant_kernel(H, S)(e128)


# speedup vs baseline: 3.0713x; 1.6056x over previous
import functools

import jax
import jax.numpy as jnp
from jax import lax
from jax.experimental import pallas as pl
from jax.experimental.pallas import tpu as pltpu
from jax.experimental.pallas import tpu_sc as plsc

_NC = 2
_NS = 16
_NW = _NC * _NS
_KG = 8


@functools.lru_cache(maxsize=None)
def _make_circulant_kernel(H, S):
  n_blocks = S // 128
  n_ug = 128 // 8
  W = S + (_KG - 1) * 128
  mesh = plsc.VectorSubcoreMesh(core_axis_name="c", subcore_axis_name="s")

  @functools.partial(
      pl.kernel,
      mesh=mesh,
      out_type=jax.ShapeDtypeStruct((H, S, S), jnp.float32),
      scratch_types=[pltpu.VMEM((8, W), jnp.float32)] * 3
      + [pltpu.SemaphoreType.DMA] * 3,
  )
  def k(e128_hbm, out_hbm, *scratch):
    bufs, sems = scratch[:3], scratch[3:]
    wid = lax.axis_index("s") * _NC + lax.axis_index("c")
    wu = lax.rem(wid, n_ug)
    wk = wid // n_ug
    u0 = pl.multiple_of(8 * wu, 8)
    wstart = pl.multiple_of(S - 128 * _KG * (wk + 1), 128)
    row0 = pl.multiple_of(128 * _KG * wk, 128)

    def issue_load(h, b):
      pltpu.async_copy(
          e128_hbm.at[h, pl.ds(u0, 8), pl.ds(wstart, W)], bufs[b], sems[b])

    def wait_load(b):
      pltpu.make_async_copy(e128_hbm.at[0, pl.ds(0, 8), pl.ds(0, W)],
                            bufs[b], sems[b]).wait()

    def issue_stores(h, b):
      for j in range(_KG):
        i_row = pl.multiple_of(row0 + 128 * j + u0, 8)
        pltpu.async_copy(bufs[b].at[:, pl.ds(128 * (_KG - 1 - j), S)],
                         out_hbm.at[h, pl.ds(i_row, 8)], sems[b])

    def drain_stores(h, b):
      for j in range(_KG):
        pltpu.make_async_copy(bufs[b].at[:, pl.ds(0, S)],
                              out_hbm.at[0, pl.ds(0, 8)], sems[b]).wait()

    issue_load(0, 0)
    if H > 1:
      issue_load(1, 1)
    for h in range(H):
      b = h % 3
      wait_load(b)
      issue_stores(h, b)
      if h + 2 < H:
        if h >= 1:
          drain_stores(h - 1, (h + 2) % 3)
        issue_load(h + 2, (h + 2) % 3)
    for h in range(max(0, H - 3), H):
      drain_stores(h, h % 3)

  return k


def _expand_body(t_ref, out_ref):
  s = t_ref.shape[-1]
  b = pltpu.roll(t_ref[0], s - 127, axis=1)
  d = 1
  while d < 128:
    b = jnp.concatenate([b, pltpu.roll(b, d, axis=1)], axis=0)
    d *= 2
  out_ref[...] = jnp.concatenate([b, b], axis=1)[None]


@functools.lru_cache(maxsize=None)
def _make_expand_kernel(H, S):
  return pl.pallas_call(
      _expand_body,
      grid=(H,),
      in_specs=[pl.BlockSpec((1, 1, S), lambda h: (h, 0, 0))],
      out_specs=pl.BlockSpec((1, 128, 2 * S), lambda h: (h, 0, 0)),
      out_shape=jax.ShapeDtypeStruct((H, 128, 2 * S), jnp.float32),
  )


def kernel(seq_len, bias_table):
  del seq_len
  S, H = bias_table.shape
  t = jnp.flip(bias_table, axis=0).T[:, None, :]
  e128 = _make_expand_kernel(H, S)(t)
  return _make_circulant_kernel(H, S)(e128)


# speedup vs baseline: 3.4075x; 1.1095x over previous
import functools

import jax
import jax.numpy as jnp
from jax import lax
from jax.experimental import pallas as pl
from jax.experimental.pallas import tpu as pltpu
from jax.experimental.pallas import tpu_sc as plsc

_NC = 2
_NS = 16
_NW = _NC * _NS


@functools.lru_cache(maxsize=None)
def _make_circulant_kernel(H, S):
  n_blocks = S // 128
  n_ug = 128 // 8
  W = 2 * S - 128
  n_units = H // 2
  mesh = plsc.VectorSubcoreMesh(core_axis_name="c", subcore_axis_name="s")

  @functools.partial(
      pl.kernel,
      mesh=mesh,
      out_type=jax.ShapeDtypeStruct((H, S, S), jnp.float32),
      scratch_types=[pltpu.VMEM((8, W), jnp.float32)] * 3
      + [pltpu.SemaphoreType.DMA] * 3,
  )
  def k(e128_hbm, out_hbm, *scratch):
    bufs, sems = scratch[:3], scratch[3:]
    wid = lax.axis_index("s") * _NC + lax.axis_index("c")
    wu = lax.rem(wid, n_ug)
    hp = wid // n_ug
    u0 = pl.multiple_of(8 * wu, 8)

    def issue_load(t, b):
      h = hp + 2 * t
      pltpu.async_copy(e128_hbm.at[h, pl.ds(u0, 8)],
                       bufs[b].at[:, pl.ds(0, S)], sems[b])
      pltpu.async_copy(e128_hbm.at[h, pl.ds(u0, 8), pl.ds(0, W - S)],
                       bufs[b].at[:, pl.ds(S, W - S)], sems[b])

    def wait_load(b):
      pltpu.make_async_copy(e128_hbm.at[0, pl.ds(0, 8)],
                            bufs[b].at[:, pl.ds(0, S)], sems[b]).wait()
      pltpu.make_async_copy(e128_hbm.at[0, pl.ds(0, 8), pl.ds(0, W - S)],
                            bufs[b].at[:, pl.ds(S, W - S)], sems[b]).wait()

    def issue_stores(t, b):
      h = hp + 2 * t
      for kb in range(n_blocks):
        i_row = pl.multiple_of(128 * kb + u0, 8)
        pltpu.async_copy(bufs[b].at[:, pl.ds(S - 128 - 128 * kb, S)],
                         out_hbm.at[h, pl.ds(i_row, 8)], sems[b])

    def drain_stores(b):
      for kb in range(n_blocks):
        pltpu.make_async_copy(bufs[b].at[:, pl.ds(0, S)],
                              out_hbm.at[0, pl.ds(0, 8)], sems[b]).wait()

    issue_load(0, 0)
    if n_units > 1:
      issue_load(1, 1)
    for t in range(n_units):
      b = t % 3
      wait_load(b)
      issue_stores(t, b)
      if t + 2 < n_units:
        if t >= 1:
          drain_stores((t + 2) % 3)
        issue_load(t + 2, (t + 2) % 3)
    for t in range(max(0, n_units - 3), n_units):
      drain_stores(t % 3)

  return k


def _expand_body(t_ref, out_ref):
  s = t_ref.shape[-1]
  b = pltpu.roll(t_ref[0], s - 127, axis=1)
  d = 1
  while d < 128:
    b = jnp.concatenate([b, pltpu.roll(b, d, axis=1)], axis=0)
    d *= 2
  out_ref[...] = b[None]


@functools.lru_cache(maxsize=None)
def _make_expand_kernel(H, S):
  return pl.pallas_call(
      _expand_body,
      grid=(H,),
      in_specs=[pl.BlockSpec((1, 1, S), lambda h: (h, 0, 0))],
      out_specs=pl.BlockSpec((1, 128, S), lambda h: (h, 0, 0)),
      out_shape=jax.ShapeDtypeStruct((H, 128, S), jnp.float32),
  )


def kernel(seq_len, bias_table):
  del seq_len
  S, H = bias_table.shape
  t = jnp.flip(bias_table, axis=0).T[:, None, :]
  e128 = _make_expand_kernel(H, S)(t)
  return _make_circulant_kernel(H, S)(e128)


# speedup vs baseline: 3.4455x; 1.0111x over previous
import functools

import jax
import jax.numpy as jnp
from jax import lax
from jax.experimental import pallas as pl
from jax.experimental.pallas import tpu as pltpu
from jax.experimental.pallas import tpu_sc as plsc

_NC = 2
_NS = 16
_NW = _NC * _NS


@functools.lru_cache(maxsize=None)
def _make_circulant_kernel(H, S):
  n_blocks = S // 128
  n_ug = 128 // 8
  W = 2 * S - 128
  n_units = H // 2
  mesh = plsc.VectorSubcoreMesh(core_axis_name="c", subcore_axis_name="s")

  @functools.partial(
      pl.kernel,
      mesh=mesh,
      out_type=jax.ShapeDtypeStruct((H, S, S), jnp.float32),
      scratch_types=[pltpu.VMEM((8, W), jnp.float32)] * 3
      + [pltpu.SemaphoreType.DMA] * 3,
  )
  def k(e128_hbm, out_hbm, *scratch):
    bufs, sems = scratch[:3], scratch[3:]
    wid = lax.axis_index("s") * _NC + lax.axis_index("c")
    wu = lax.rem(wid, n_ug)
    hp = wid // n_ug
    u0 = pl.multiple_of(8 * wu, 8)

    def issue_load(t, b):
      h = hp + 2 * t
      pltpu.async_copy(e128_hbm.at[h, pl.ds(u0, 8)],
                       bufs[b].at[:, pl.ds(0, S)], sems[b])
      pltpu.async_copy(e128_hbm.at[h, pl.ds(u0, 8), pl.ds(0, W - S)],
                       bufs[b].at[:, pl.ds(S, W - S)], sems[b])

    def wait_load(b):
      pltpu.make_async_copy(e128_hbm.at[0, pl.ds(0, 8)],
                            bufs[b].at[:, pl.ds(0, S)], sems[b]).wait()
      pltpu.make_async_copy(e128_hbm.at[0, pl.ds(0, 8), pl.ds(0, W - S)],
                            bufs[b].at[:, pl.ds(S, W - S)], sems[b]).wait()

    def issue_stores(t, b):
      h = hp + 2 * t
      for kb in range(n_blocks):
        i_row = pl.multiple_of(128 * kb + u0, 8)
        pltpu.async_copy(bufs[b].at[:, pl.ds(S - 128 - 128 * kb, S)],
                         out_hbm.at[h, pl.ds(i_row, 8)], sems[b])

    def drain_stores(b):
      for kb in range(n_blocks):
        pltpu.make_async_copy(bufs[b].at[:, pl.ds(0, S)],
                              out_hbm.at[0, pl.ds(0, 8)], sems[b]).wait()

    issue_load(0, 0)
    if n_units > 1:
      issue_load(1, 1)
    for t in range(n_units):
      b = t % 3
      wait_load(b)
      issue_stores(t, b)
      if t + 2 < n_units:
        if t >= 1:
          drain_stores((t + 2) % 3)
        issue_load(t + 2, (t + 2) % 3)
    for t in range(max(0, n_units - 3), n_units):
      drain_stores(t % 3)

  return k


def _expand_body(t_ref, out_ref):
  s = t_ref.shape[-1]
  v = pltpu.roll(t_ref[0], s - 127, axis=1)
  x = jnp.broadcast_to(v, (128, s))
  out_ref[...] = pltpu.roll(x, 0, axis=1, stride=1, stride_axis=0)[None]


@functools.lru_cache(maxsize=None)
def _make_expand_kernel(H, S):
  return pl.pallas_call(
      _expand_body,
      grid=(H,),
      in_specs=[pl.BlockSpec((1, 1, S), lambda h: (h, 0, 0))],
      out_specs=pl.BlockSpec((1, 128, S), lambda h: (h, 0, 0)),
      out_shape=jax.ShapeDtypeStruct((H, 128, S), jnp.float32),
  )


def kernel(seq_len, bias_table):
  del seq_len
  S, H = bias_table.shape
  t = jnp.flip(bias_table, axis=0).T[:, None, :]
  e128 = _make_expand_kernel(H, S)(t)
  return _make_circulant_kernel(H, S)(e128)
